# baseline (speedup 1.0000x reference)
import functools

import jax
import jax.numpy as jnp
from jax import lax
from jax.experimental import pallas as pl
from jax.experimental.pallas import tpu as pltpu
from jax.experimental.pallas import tpu_sc as plsc

_S = 4


@functools.cache
def _gunpool_sc(B, V, E, d, NC, NS):
    NW = NC * NS
    EW = E // NW
    assert E % NW == 0
    K = 1
    for cand in (128, 120, 112, 104, 96, 88, 80, 72, 64, 56, 48, 40, 32, 24, 16, 8):
        if EW % cand == 0:
            K = cand
            break
    CPB = EW // K
    N = B * CPB
    assert N >= 12
    VO = V + E

    mesh = plsc.VectorSubcoreMesh(core_axis_name="c", subcore_axis_name="s")

    @functools.partial(
        pl.kernel,
        out_type=jax.ShapeDtypeStruct((B * VO, d), jnp.float32),
        mesh=mesh,
        scratch_types=(
            [pltpu.VMEM((2, K), jnp.int32) for _ in range(_S)]
            + [pltpu.VMEM((K, d), jnp.float32) for _ in range(_S)]
            + [pltpu.SemaphoreType.DMA] * (4 * _S)
        ),
    )
    def k(x_hbm, idxp_hbm, out_hbm, *scr):
        idxs = list(scr[:_S])
        ras = list(scr[_S:2 * _S])
        sems = list(scr[2 * _S:])
        isem = sems[0:_S]
        gsa = sems[_S:2 * _S]
        gsb = sems[2 * _S:3 * _S]
        osem = sems[3 * _S:4 * _S]
        wid = lax.axis_index("s") * NC + lax.axis_index("c")

        def split(c):
            if B == 2:
                b = (c >= CPB).astype(jnp.int32) if not isinstance(c, int) \
                    else int(c >= CPB)
            else:
                b = c // CPB
            return b, c - b * CPB

        def fire_idx(c, s):
            b, local = split(c)
            row = b * (E // K) + wid * CPB + local
            pltpu.async_copy(idxp_hbm.at[row], idxs[s], isem[s])

        def drain_idx(s):
            pltpu.make_async_copy(idxp_hbm.at[0], idxs[s], isem[s]).wait()

        def fire_a(s):
            pltpu.async_copy(x_hbm.at[idxs[s].at[0]], ras[s], gsa[s])

        def drain_a(s):
            pltpu.make_async_copy(x_hbm.at[idxs[s].at[0]], ras[s], gsa[s]).wait()

        def fire_b(s):
            pltpu.async_copy(x_hbm.at[idxs[s].at[1]], ras[s], gsb[s], add=True)

        def drain_b(s):
            pltpu.make_async_copy(x_hbm.at[idxs[s].at[1]], ras[s], gsb[s]).wait()

        def compute(s):
            def body(i, _):
                for j in range(d // 16):
                    sl = pl.ds(j * 16, 16)
                    ras[s][i, sl] = ras[s][i, sl] * 0.5
                return 0
            lax.fori_loop(0, K, body, 0)

        def fire_out(c, s):
            b, local = split(c)
            o0 = b * VO + V + wid * EW + local * K
            pltpu.async_copy(ras[s], out_hbm.at[pl.ds(o0, K)], osem[s])

        def drain_out(s):
            pltpu.make_async_copy(ras[s], out_hbm.at[pl.ds(0, K)], osem[s]).wait()

        def iter_body(c, p, b_next=True, a_2=True, o_wait=True, i_3=True):
            q1, q2, q3 = (p + 1) % _S, (p + 2) % _S, (p + 3) % _S
            if b_next:
                drain_a(q1)
                fire_b(q1)
            drain_b(p)
            if a_2:
                drain_idx(q2)
                if o_wait:
                    drain_out(q2)
                fire_a(q2)
            if i_3:
                fire_idx(c + 3, q3)
            compute(p)
            fire_out(c, p)

        CPY = 320
        n_full = V // CPY
        rem = V - n_full * CPY
        for b in range(B):
            @pl.when(wid < n_full)
            def _():
                r0 = wid * CPY
                pltpu.sync_copy(x_hbm.at[pl.ds(b * V + r0, CPY)],
                                out_hbm.at[pl.ds(b * VO + r0, CPY)])
            if rem:
                @pl.when(wid == n_full)
                def _():
                    r0 = n_full * CPY
                    pltpu.sync_copy(x_hbm.at[pl.ds(b * V + r0, rem)],
                                    out_hbm.at[pl.ds(b * VO + r0, rem)])

        fire_idx(0, 0)
        fire_idx(1, 1)
        fire_idx(2, 2)
        drain_idx(0)
        fire_a(0)
        drain_idx(1)
        fire_a(1)
        drain_a(0)
        fire_b(0)

        iter_body(0, 0, o_wait=False)
        iter_body(1, 1, o_wait=False)
        iter_body(2, 2)
        iter_body(3, 3)

        n_main = (N - 4 - 6) // _S * _S
        def step(i, _):
            c0 = 4 + _S * i
            for p in range(_S):
                iter_body(c0 + p, p)
            return 0
        lax.fori_loop(0, n_main // _S, step, 0)

        for c in range(4 + n_main, N):
            p = c % _S
            iter_body(c, p,
                      b_next=(c + 1 < N),
                      a_2=(c + 2 < N),
                      i_3=(c + 3 < N))

        for s in range(_S):
            drain_out(s)

    return k


def kernel(x, edge_index):
    B, V, d = x.shape
    E = edge_index.shape[1]
    idx = edge_index.astype(jnp.int32)
    offs = (jnp.arange(B, dtype=jnp.int32) * V)[:, None]
    src_all = (idx[0][None, :] + offs).reshape(-1)
    dst_all = (idx[1][None, :] + offs).reshape(-1)
    x2 = x.reshape(B * V, d)
    info = plsc.get_sparse_core_info()
    NC, NS = info.num_cores, info.num_subcores
    NW = NC * NS
    EW = E // NW
    K = 1
    for cand in (128, 120, 112, 104, 96, 88, 80, 72, 64, 56, 48, 40, 32, 24, 16, 8):
        if EW % cand == 0:
            K = cand
            break
    idx_packed = jnp.stack(
        [src_all.reshape(-1, K), dst_all.reshape(-1, K)], axis=1)
    out = _gunpool_sc(B, V, E, d, NC, NS)(x2, idx_packed)
    return out.reshape(B, V + E, d)
